# baseline (speedup 1.0000x reference)
import functools

import jax
import jax.numpy as jnp
from jax import lax
from jax.experimental import pallas as pl
from jax.experimental.pallas import tpu as pltpu
from jax.experimental.pallas import tpu_sc as plsc

B = 128
V = 100000
TOPK = 32
HID = 128
OUT = 128

NC = 2
NS = 16
NW = NC * NS
ROWS_PER_W = B // NW
LANES = 16

NCHUNK = 5
CH = V // NCHUNK
GROUP = 10
NG_C = CH // (GROUP * LANES)
NQUAD = NG_C // 4

_NEG = float("-inf")


def _sortd(x):
  return lax.rev(jnp.sort(x), (0,))


def _any16(mask):
  cnt = plsc.all_reduce_population_count(mask)
  return cnt[0] > 0


def _splat_lane(x, lane):
  idx = jnp.full((LANES, 1), lane, jnp.int32)
  dnums = lax.GatherDimensionNumbers(
      offset_dims=(), collapsed_slice_dims=(0,), start_index_map=(0,))
  return lax.gather(x, idx, dnums, (1,),
                    mode=lax.GatherScatterMode.PROMISE_IN_BOUNDS)


def _merge32(a, b, v):
  vs = _sortd(v)
  m = jnp.maximum(b, lax.rev(vs, (0,)))
  ms = _sortd(m)
  x = jnp.maximum(a, lax.rev(ms, (0,)))
  y = jnp.minimum(a, lax.rev(ms, (0,)))
  return _sortd(x), _sortd(y)


def _maybe_merge(v, carry):
  a, b, t = carry

  def do(_):
    a2, b2 = _merge32(a, b, v)
    t2 = jnp.maximum(t, _splat_lane(b2, LANES - 1))
    return a2, b2, t2

  return lax.cond(_any16(v >= t), do, lambda _: (a, b, t), None)


def _drill_group(buf, gbase, carry):
  def vcheck(j, c):
    v = buf[pl.ds((gbase * GROUP + j) * LANES, LANES)]
    return _maybe_merge(v, c)
  return lax.fori_loop(0, GROUP, vcheck, carry)


def _sc_topk(logits):
  mesh = plsc.VectorSubcoreMesh(
      core_axis_name="c", subcore_axis_name="s", num_cores=NC, num_subcores=NS)

  @functools.partial(
      pl.kernel,
      out_type=jax.ShapeDtypeStruct((B, TOPK), jnp.float32),
      mesh=mesh,
      scratch_types=[
          pltpu.VMEM((V,), jnp.float32),
          pltpu.VMEM((NG_C * NCHUNK * LANES,), jnp.float32),
          pltpu.VMEM((TOPK,), jnp.float32),
      ] + [pltpu.SemaphoreType.DMA] * NCHUNK,
      compiler_params=pltpu.CompilerParams(needs_layout_passes=False),
  )
  def k(logits_hbm, out_hbm, buf, gmax, obuf, *sems):
    wid = lax.axis_index("s") * NC + lax.axis_index("c")

    def row_body(r, _):
      row = wid * ROWS_PER_W + r
      copies = [
          pltpu.make_async_copy(
              logits_hbm.at[pl.ds(row * V + c * CH, CH)],
              buf.at[pl.ds(c * CH, CH)],
              sems[c])
          for c in range(NCHUNK)
      ]
      for c in range(NCHUNK):
        copies[c].start()

      neg = jnp.full((LANES,), _NEG, jnp.float32)
      state = (neg, neg, neg)
      seed = (neg, neg)

      for c in range(NCHUNK):
        copies[c].wait()

        def g_body(g, sc, c=c):
          m1, m2 = sc
          gi = c * NG_C + g
          vecs = [buf[pl.ds((gi * GROUP + j) * LANES, LANES)]
                  for j in range(GROUP)]
          while len(vecs) > 1:
            vecs = [jnp.maximum(vecs[i], vecs[i + 1])
                    for i in range(0, len(vecs) - 1, 2)] + (
                        [vecs[-1]] if len(vecs) % 2 else [])
          gmx = vecs[0]
          hi = jnp.maximum(m1, gmx)
          lo = jnp.minimum(m1, gmx)
          gmax[pl.ds(gi * LANES, LANES)] = gmx
          return hi, jnp.maximum(m2, lo)

        seed = lax.fori_loop(0, NG_C, g_body, seed)
        a, b, t = state
        t = jnp.maximum(t, _splat_lane(_sortd(seed[1]), LANES - 1))
        state = (a, b, t)

        def quad_body(q, carry, c=c):
          gvi = c * NG_C + q * 4
          g0 = gmax[pl.ds(gvi * LANES, LANES)]
          g1 = gmax[pl.ds((gvi + 1) * LANES, LANES)]
          g2 = gmax[pl.ds((gvi + 2) * LANES, LANES)]
          g3 = gmax[pl.ds((gvi + 3) * LANES, LANES)]
          gm = jnp.maximum(jnp.maximum(g0, g1), jnp.maximum(g2, g3))

          def drill(cin):
            def gcheck(k2, cc):
              gv = gmax[pl.ds((gvi + k2) * LANES, LANES)]

              def drill2(c3):
                return _drill_group(buf, gvi + k2, c3)

              return lax.cond(_any16(gv >= cc[2]), drill2, lambda c3: c3, cc)
            return lax.fori_loop(0, 4, gcheck, cin)

          return lax.cond(_any16(gm >= carry[2]), drill, lambda cin: cin, carry)

        state = lax.fori_loop(0, NQUAD, quad_body, state)
        gvi = c * NG_C + NQUAD * 4
        gv = gmax[pl.ds(gvi * LANES, LANES)]
        state = lax.cond(
            _any16(gv >= state[2]),
            lambda cc: _drill_group(buf, gvi, cc),
            lambda cc: cc, state)

      a, b, _ = state
      obuf[pl.ds(0, LANES)] = a
      obuf[pl.ds(LANES, LANES)] = b
      pltpu.sync_copy(obuf, out_hbm.at[row])
      return 0

    lax.fori_loop(0, ROWS_PER_W, row_body, 0)

  return k(logits.reshape(B * V))


def _tc_head(x, ln_w, ln_b, w1, b1, w2, b2):
  def body(x_ref, lnw_ref, lnb_ref, w1_ref, b1_ref, w2_ref, b2_ref, o_ref):
    xv = x_ref[...]
    mean = jnp.mean(xv, axis=-1, keepdims=True)
    var = jnp.mean((xv - mean) ** 2, axis=-1, keepdims=True)
    xn = (xv - mean) * lax.rsqrt(var + 1e-5) * lnw_ref[...] + lnb_ref[...]
    h = jnp.dot(xn, w1_ref[...], preferred_element_type=jnp.float32) + b1_ref[...]
    h = 0.5 * h * (1.0 + lax.erf(h * jnp.float32(0.7071067811865476)))
    o_ref[...] = jnp.dot(h, w2_ref[...], preferred_element_type=jnp.float32) + b2_ref[...]

  return pl.pallas_call(
      body,
      out_shape=jax.ShapeDtypeStruct((B, OUT), jnp.float32),
  )(x, ln_w.reshape(1, TOPK), ln_b.reshape(1, TOPK),
    w1, b1.reshape(1, HID), w2, b2.reshape(1, OUT))


def kernel(logits, ln_w, ln_b, W1, b1, W2, b2):
  topk = _sc_topk(logits)
  return _tc_head(topk, ln_w, ln_b, W1, b1, W2, b2)


# speedup vs baseline: 1.5592x; 1.5592x over previous
import functools

import jax
import jax.numpy as jnp
from jax import lax
from jax.experimental import pallas as pl
from jax.experimental.pallas import tpu as pltpu
from jax.experimental.pallas import tpu_sc as plsc

B = 128
V = 100000
TOPK = 32
HID = 128
OUT = 128

NC = 2
NS = 16
NW = NC * NS
ROWS_PER_W = B // NW
LANES = 16

NCHUNK = 0
GROUP = 10
NG = V // (GROUP * LANES)
NQUAD = NG // 4

_NEG = float("-inf")


def _sortd(x):
  return lax.rev(jnp.sort(x), (0,))


def _any16(mask):
  cnt = plsc.all_reduce_population_count(mask)
  return cnt[0] > 0


def _splat_lane(x, lane):
  idx = jnp.full((LANES, 1), lane, jnp.int32)
  dnums = lax.GatherDimensionNumbers(
      offset_dims=(), collapsed_slice_dims=(0,), start_index_map=(0,))
  return lax.gather(x, idx, dnums, (1,),
                    mode=lax.GatherScatterMode.PROMISE_IN_BOUNDS)


def _merge32(a, b, v):
  vs = _sortd(v)
  m = jnp.maximum(b, lax.rev(vs, (0,)))
  ms = _sortd(m)
  x = jnp.maximum(a, lax.rev(ms, (0,)))
  y = jnp.minimum(a, lax.rev(ms, (0,)))
  return _sortd(x), _sortd(y)


def _maybe_merge(v, carry):
  a, b, t = carry

  def do(_):
    a2, b2 = _merge32(a, b, v)
    t2 = jnp.maximum(t, _splat_lane(b2, LANES - 1))
    return a2, b2, t2

  return lax.cond(_any16(v >= t), do, lambda _: (a, b, t), None)


def _drill_group(buf, gbase, carry):
  def vcheck(j, c):
    v = buf[pl.ds((gbase * GROUP + j) * LANES, LANES)]
    return _maybe_merge(v, c)
  return lax.fori_loop(0, GROUP, vcheck, carry)


def _sc_topk(logits):
  mesh = plsc.VectorSubcoreMesh(
      core_axis_name="c", subcore_axis_name="s", num_cores=NC, num_subcores=NS)

  @functools.partial(
      pl.kernel,
      out_type=jax.ShapeDtypeStruct((B, TOPK), jnp.float32),
      mesh=mesh,
      scratch_types=[
          pltpu.VMEM((V,), jnp.float32),
          pltpu.VMEM((NG * LANES,), jnp.float32),
          pltpu.VMEM((TOPK,), jnp.float32),
      ],
      compiler_params=pltpu.CompilerParams(needs_layout_passes=False),
  )
  def k(logits_hbm, out_hbm, buf, gmax, obuf):
    wid = lax.axis_index("s") * NC + lax.axis_index("c")

    def row_body(r, _):
      row = wid * ROWS_PER_W + r
      pltpu.sync_copy(logits_hbm.at[row], buf)

      neg = jnp.full((LANES,), _NEG, jnp.float32)

      @plsc.parallel_loop(0, NG, unroll=2, carry=(neg, neg))
      def seed(g, sc):
        m1, m2 = sc
        vecs = [buf[pl.ds((g * GROUP + j) * LANES, LANES)]
                for j in range(GROUP)]
        while len(vecs) > 1:
          vecs = [jnp.maximum(vecs[i], vecs[i + 1])
                  for i in range(0, len(vecs) - 1, 2)] + (
                      [vecs[-1]] if len(vecs) % 2 else [])
        gmx = vecs[0]
        hi = jnp.maximum(m1, gmx)
        lo = jnp.minimum(m1, gmx)
        gmax[pl.ds(g * LANES, LANES)] = gmx
        return hi, jnp.maximum(m2, lo)

      t0 = jnp.maximum(neg, _splat_lane(_sortd(seed[1]), LANES - 1))
      state = (neg, neg, t0)

      def quad_body(q, carry):
        gvi = q * 4
        g0 = gmax[pl.ds(gvi * LANES, LANES)]
        g1 = gmax[pl.ds((gvi + 1) * LANES, LANES)]
        g2 = gmax[pl.ds((gvi + 2) * LANES, LANES)]
        g3 = gmax[pl.ds((gvi + 3) * LANES, LANES)]
        gm = jnp.maximum(jnp.maximum(g0, g1), jnp.maximum(g2, g3))

        def drill(cin):
          def gcheck(k2, cc):
            gv = gmax[pl.ds((gvi + k2) * LANES, LANES)]

            def drill2(c3):
              return _drill_group(buf, gvi + k2, c3)

            return lax.cond(_any16(gv >= cc[2]), drill2, lambda c3: c3, cc)
          return lax.fori_loop(0, 4, gcheck, cin)

        return lax.cond(_any16(gm >= carry[2]), drill, lambda cin: cin, carry)

      state = lax.fori_loop(0, NQUAD, quad_body, state)
      gvi = NQUAD * 4
      gv = gmax[pl.ds(gvi * LANES, LANES)]
      state = lax.cond(
          _any16(gv >= state[2]),
          lambda cc: _drill_group(buf, gvi, cc),
          lambda cc: cc, state)

      a, b, _ = state
      obuf[pl.ds(0, LANES)] = a
      obuf[pl.ds(LANES, LANES)] = b
      pltpu.sync_copy(obuf, out_hbm.at[row])
      return 0

    lax.fori_loop(0, ROWS_PER_W, row_body, 0)

  return k(logits)


def _tc_head(x, ln_w, ln_b, w1, b1, w2, b2):
  def body(x_ref, lnw_ref, lnb_ref, w1_ref, b1_ref, w2_ref, b2_ref, o_ref):
    xv = x_ref[...]
    mean = jnp.mean(xv, axis=-1, keepdims=True)
    var = jnp.mean((xv - mean) ** 2, axis=-1, keepdims=True)
    xn = (xv - mean) * lax.rsqrt(var + 1e-5) * lnw_ref[...] + lnb_ref[...]
    h = jnp.dot(xn, w1_ref[...], preferred_element_type=jnp.float32) + b1_ref[...]
    h = 0.5 * h * (1.0 + lax.erf(h * jnp.float32(0.7071067811865476)))
    o_ref[...] = jnp.dot(h, w2_ref[...], preferred_element_type=jnp.float32) + b2_ref[...]

  return pl.pallas_call(
      body,
      out_shape=jax.ShapeDtypeStruct((B, OUT), jnp.float32),
  )(x, ln_w.reshape(1, TOPK), ln_b.reshape(1, TOPK),
    w1, b1.reshape(1, HID), w2, b2.reshape(1, OUT))


def kernel(logits, ln_w, ln_b, W1, b1, W2, b2):
  topk = _sc_topk(logits)
  return _tc_head(topk, ln_w, ln_b, W1, b1, W2, b2)


# speedup vs baseline: 1.5605x; 1.0009x over previous
import functools

import jax
import jax.numpy as jnp
from jax import lax
from jax.experimental import pallas as pl
from jax.experimental.pallas import tpu as pltpu
from jax.experimental.pallas import tpu_sc as plsc

B = 128
V = 100000
TOPK = 32
HID = 128
OUT = 128

NC = 2
NS = 16
NW = NC * NS
ROWS_PER_W = B // NW
LANES = 16

NCHUNK = 0
GROUP = 10
NG = V // (GROUP * LANES)
NQUAD = NG // 4

_NEG = float("-inf")


def _sortd(x):
  return lax.rev(jnp.sort(x), (0,))


def _any16(mask):
  cnt = plsc.all_reduce_population_count(mask)
  return cnt[0] > 0


def _splat_lane(x, lane):
  idx = jnp.full((LANES, 1), lane, jnp.int32)
  dnums = lax.GatherDimensionNumbers(
      offset_dims=(), collapsed_slice_dims=(0,), start_index_map=(0,))
  return lax.gather(x, idx, dnums, (1,),
                    mode=lax.GatherScatterMode.PROMISE_IN_BOUNDS)


def _merge32(a, b, v):
  vs = _sortd(v)
  m = jnp.maximum(b, lax.rev(vs, (0,)))
  ms = _sortd(m)
  x = jnp.maximum(a, lax.rev(ms, (0,)))
  y = jnp.minimum(a, lax.rev(ms, (0,)))
  return _sortd(x), _sortd(y)


def _maybe_merge(v, carry):
  a, b, t = carry

  def do(_):
    a2, b2 = _merge32(a, b, v)
    t2 = jnp.maximum(t, _splat_lane(b2, LANES - 1))
    return a2, b2, t2

  return lax.cond(_any16(v >= t), do, lambda _: (a, b, t), None)


def _drill_group(buf, gbase, carry):
  c = carry
  for j in range(GROUP):
    v = buf[pl.ds((gbase * GROUP + j) * LANES, LANES)]
    c = _maybe_merge(v, c)
  return c


def _sc_topk(logits):
  mesh = plsc.VectorSubcoreMesh(
      core_axis_name="c", subcore_axis_name="s", num_cores=NC, num_subcores=NS)

  @functools.partial(
      pl.kernel,
      out_type=jax.ShapeDtypeStruct((B, TOPK), jnp.float32),
      mesh=mesh,
      scratch_types=[
          pltpu.VMEM((V,), jnp.float32),
          pltpu.VMEM((NG * LANES,), jnp.float32),
          pltpu.VMEM((TOPK,), jnp.float32),
      ],
      compiler_params=pltpu.CompilerParams(needs_layout_passes=False),
  )
  def k(logits_hbm, out_hbm, buf, gmax, obuf):
    wid = lax.axis_index("s") * NC + lax.axis_index("c")

    def row_body(r, _):
      row = wid * ROWS_PER_W + r
      pltpu.sync_copy(logits_hbm.at[row], buf)

      neg = jnp.full((LANES,), _NEG, jnp.float32)

      @plsc.parallel_loop(0, NG, unroll=2, carry=(neg, neg))
      def seed(g, sc):
        m1, m2 = sc
        vecs = [buf[pl.ds((g * GROUP + j) * LANES, LANES)]
                for j in range(GROUP)]
        while len(vecs) > 1:
          vecs = [jnp.maximum(vecs[i], vecs[i + 1])
                  for i in range(0, len(vecs) - 1, 2)] + (
                      [vecs[-1]] if len(vecs) % 2 else [])
        gmx = vecs[0]
        hi = jnp.maximum(m1, gmx)
        lo = jnp.minimum(m1, gmx)
        gmax[pl.ds(g * LANES, LANES)] = gmx
        return hi, jnp.maximum(m2, lo)

      t0 = jnp.maximum(neg, _splat_lane(_sortd(seed[1]), LANES - 1))
      state = (neg, neg, t0)

      def quad_body(q, carry):
        gvi = q * 4
        g0 = gmax[pl.ds(gvi * LANES, LANES)]
        g1 = gmax[pl.ds((gvi + 1) * LANES, LANES)]
        g2 = gmax[pl.ds((gvi + 2) * LANES, LANES)]
        g3 = gmax[pl.ds((gvi + 3) * LANES, LANES)]
        gm = jnp.maximum(jnp.maximum(g0, g1), jnp.maximum(g2, g3))

        def drill(cin):
          def gcheck(k2, cc):
            gv = gmax[pl.ds((gvi + k2) * LANES, LANES)]

            def drill2(c3):
              return _drill_group(buf, gvi + k2, c3)

            return lax.cond(_any16(gv >= cc[2]), drill2, lambda c3: c3, cc)
          return lax.fori_loop(0, 4, gcheck, cin)

        return lax.cond(_any16(gm >= carry[2]), drill, lambda cin: cin, carry)

      state = lax.fori_loop(0, NQUAD, quad_body, state)
      gvi = NQUAD * 4
      gv = gmax[pl.ds(gvi * LANES, LANES)]
      state = lax.cond(
          _any16(gv >= state[2]),
          lambda cc: _drill_group(buf, gvi, cc),
          lambda cc: cc, state)

      a, b, _ = state
      obuf[pl.ds(0, LANES)] = a
      obuf[pl.ds(LANES, LANES)] = b
      pltpu.sync_copy(obuf, out_hbm.at[row])
      return 0

    lax.fori_loop(0, ROWS_PER_W, row_body, 0)

  return k(logits)


def _tc_head(x, ln_w, ln_b, w1, b1, w2, b2):
  def body(x_ref, lnw_ref, lnb_ref, w1_ref, b1_ref, w2_ref, b2_ref, o_ref):
    xv = x_ref[...]
    mean = jnp.mean(xv, axis=-1, keepdims=True)
    var = jnp.mean((xv - mean) ** 2, axis=-1, keepdims=True)
    xn = (xv - mean) * lax.rsqrt(var + 1e-5) * lnw_ref[...] + lnb_ref[...]
    h = jnp.dot(xn, w1_ref[...], preferred_element_type=jnp.float32) + b1_ref[...]
    h = 0.5 * h * (1.0 + lax.erf(h * jnp.float32(0.7071067811865476)))
    o_ref[...] = jnp.dot(h, w2_ref[...], preferred_element_type=jnp.float32) + b2_ref[...]

  return pl.pallas_call(
      body,
      out_shape=jax.ShapeDtypeStruct((B, OUT), jnp.float32),
  )(x, ln_w.reshape(1, TOPK), ln_b.reshape(1, TOPK),
    w1, b1.reshape(1, HID), w2, b2.reshape(1, OUT))


def kernel(logits, ln_w, ln_b, W1, b1, W2, b2):
  topk = _sc_topk(logits)
  return _tc_head(topk, ln_w, ln_b, W1, b1, W2, b2)


# speedup vs baseline: 2.6412x; 1.6925x over previous
import functools

import jax
import jax.numpy as jnp
from jax import lax
from jax.experimental import pallas as pl
from jax.experimental.pallas import tpu as pltpu
from jax.experimental.pallas import tpu_sc as plsc

B = 128
V = 100000
TOPK = 32
HID = 128
OUT = 128

NC = 2
NS = 16
NW = NC * NS
ROWS_PER_W = B // NW
LANES = 16

NCHUNK = 0
GROUP = 10
NG = V // (GROUP * LANES)
NQUAD = NG // 4

_NEG = float("-inf")


def _sortd(x):
  return lax.rev(jnp.sort(x), (0,))


def _any16(mask):
  cnt = plsc.all_reduce_population_count(mask)
  return cnt[0] > 0


def _splat_lane(x, lane):
  idx = jnp.full((LANES, 1), lane, jnp.int32)
  dnums = lax.GatherDimensionNumbers(
      offset_dims=(), collapsed_slice_dims=(0,), start_index_map=(0,))
  return lax.gather(x, idx, dnums, (1,),
                    mode=lax.GatherScatterMode.PROMISE_IN_BOUNDS)


def _merge32(a, b, v):
  vs = _sortd(v)
  m = jnp.maximum(b, lax.rev(vs, (0,)))
  ms = _sortd(m)
  x = jnp.maximum(a, lax.rev(ms, (0,)))
  y = jnp.minimum(a, lax.rev(ms, (0,)))
  return _sortd(x), _sortd(y)


def _maybe_merge(v, carry):
  a, b, t = carry

  def do(_):
    a2, b2 = _merge32(a, b, v)
    t2 = jnp.maximum(t, _splat_lane(b2, LANES - 1))
    return a2, b2, t2

  return lax.cond(_any16(v >= t), do, lambda _: (a, b, t), None)


def _sc_topk(logits):
  mesh = plsc.VectorSubcoreMesh(
      core_axis_name="c", subcore_axis_name="s", num_cores=NC, num_subcores=NS)

  @functools.partial(
      pl.kernel,
      out_type=jax.ShapeDtypeStruct((B, TOPK), jnp.float32),
      mesh=mesh,
      scratch_types=[
          pltpu.VMEM((V,), jnp.float32),
          pltpu.VMEM((NG * LANES,), jnp.float32),
          pltpu.VMEM((NG * LANES + 2 * LANES,), jnp.int32),
          pltpu.VMEM((TOPK,), jnp.float32),
          pltpu.SMEM((1,), jnp.int32),
      ],
      compiler_params=pltpu.CompilerParams(needs_layout_passes=False),
  )
  def k(logits_hbm, out_hbm, buf, gmax, wl, obuf, cnt_ref):
    wid = lax.axis_index("s") * NC + lax.axis_index("c")

    def row_body(r, _):
      row = wid * ROWS_PER_W + r
      pltpu.sync_copy(logits_hbm.at[row], buf)

      neg = jnp.full((LANES,), _NEG, jnp.float32)

      @plsc.parallel_loop(0, NG, unroll=2, carry=(neg, neg))
      def seed(g, sc):
        m1, m2 = sc
        vecs = [buf[pl.ds((g * GROUP + j) * LANES, LANES)]
                for j in range(GROUP)]
        while len(vecs) > 1:
          vecs = [jnp.maximum(vecs[i], vecs[i + 1])
                  for i in range(0, len(vecs) - 1, 2)] + (
                      [vecs[-1]] if len(vecs) % 2 else [])
        gmx = vecs[0]
        hi = jnp.maximum(m1, gmx)
        lo = jnp.minimum(m1, gmx)
        gmax[pl.ds(g * LANES, LANES)] = gmx
        return hi, jnp.maximum(m2, lo)

      t0 = jnp.maximum(neg, _splat_lane(_sortd(seed[1]), LANES - 1))

      cnt_ref[0] = 0
      iota = lax.iota(jnp.int32, LANES)

      def compress_one(gv_idx):
        gv = gmax[pl.ds(gv_idx * LANES, LANES)]
        mask = gv >= t0
        ids = gv_idx * LANES + iota
        c = cnt_ref[0]
        plsc.store_compressed(wl.at[pl.ds(c, LANES)], ids, mask=mask)
        cnt_ref[0] = c + plsc.all_reduce_population_count(mask)[0]

      def quad_body(q, _):
        gvi = q * 4
        g0 = gmax[pl.ds(gvi * LANES, LANES)]
        g1 = gmax[pl.ds((gvi + 1) * LANES, LANES)]
        g2 = gmax[pl.ds((gvi + 2) * LANES, LANES)]
        g3 = gmax[pl.ds((gvi + 3) * LANES, LANES)]
        gm = jnp.maximum(jnp.maximum(g0, g1), jnp.maximum(g2, g3))

        def hit(_2):
          for k2 in range(4):
            compress_one(gvi + k2)
          return 0

        return lax.cond(_any16(gm >= t0), hit, lambda _2: 0, 0)

      lax.fori_loop(0, NQUAD, quad_body, 0)
      compress_one(NQUAD * 4)
      n = cnt_ref[0]
      wl[pl.ds(n, LANES)] = jnp.full((LANES,), -1, jnp.int32)

      validc = iota < GROUP

      def drain_block(i, st):
        wvec = wl[pl.ds(i * LANES, LANES)]

        for j in range(LANES):
          e = wvec[j]
          g = lax.shift_right_logical(e, 4)
          lane = lax.bitwise_and(e, 15)
          base = g * (GROUP * LANES) + lane
          vmask = jnp.logical_and(validc, e >= 0)
          idx = jnp.where(vmask, base + iota * LANES, 0)
          v = plsc.load_gather(buf, [idx])
          v = jnp.where(vmask, v, _NEG)
          st = _maybe_merge(v, st)
        return st

      nblk = (n + LANES - 1) // LANES
      state = lax.fori_loop(0, nblk, drain_block, (neg, neg, t0))

      a, b, _ = state
      obuf[pl.ds(0, LANES)] = a
      obuf[pl.ds(LANES, LANES)] = b
      pltpu.sync_copy(obuf, out_hbm.at[row])
      return 0

    lax.fori_loop(0, ROWS_PER_W, row_body, 0)

  return k(logits)


def _tc_head(x, ln_w, ln_b, w1, b1, w2, b2):
  def body(x_ref, lnw_ref, lnb_ref, w1_ref, b1_ref, w2_ref, b2_ref, o_ref):
    xv = x_ref[...]
    mean = jnp.mean(xv, axis=-1, keepdims=True)
    var = jnp.mean((xv - mean) ** 2, axis=-1, keepdims=True)
    xn = (xv - mean) * lax.rsqrt(var + 1e-5) * lnw_ref[...] + lnb_ref[...]
    h = jnp.dot(xn, w1_ref[...], preferred_element_type=jnp.float32) + b1_ref[...]
    h = 0.5 * h * (1.0 + lax.erf(h * jnp.float32(0.7071067811865476)))
    o_ref[...] = jnp.dot(h, w2_ref[...], preferred_element_type=jnp.float32) + b2_ref[...]

  return pl.pallas_call(
      body,
      out_shape=jax.ShapeDtypeStruct((B, OUT), jnp.float32),
  )(x, ln_w.reshape(1, TOPK), ln_b.reshape(1, TOPK),
    w1, b1.reshape(1, HID), w2, b2.reshape(1, OUT))


def kernel(logits, ln_w, ln_b, W1, b1, W2, b2):
  topk = _sc_topk(logits)
  return _tc_head(topk, ln_w, ln_b, W1, b1, W2, b2)


# speedup vs baseline: 2.8553x; 1.0810x over previous
import functools

import jax
import jax.numpy as jnp
from jax import lax
from jax.experimental import pallas as pl
from jax.experimental.pallas import tpu as pltpu
from jax.experimental.pallas import tpu_sc as plsc

B = 128
V = 100000
TOPK = 32
HID = 128
OUT = 128

NC = 2
NS = 16
NW = NC * NS
ROWS_PER_W = B // NW
LANES = 16

NCHUNK = 0
GROUP = 10
NG = V // (GROUP * LANES)
NQUAD = NG // 4
CAND_CAP = 2048

_NEG = float("-inf")


def _sortd(x):
  return lax.rev(jnp.sort(x), (0,))


def _any16(mask):
  cnt = plsc.all_reduce_population_count(mask)
  return cnt[0] > 0


def _splat_lane(x, lane):
  idx = jnp.full((LANES, 1), lane, jnp.int32)
  dnums = lax.GatherDimensionNumbers(
      offset_dims=(), collapsed_slice_dims=(0,), start_index_map=(0,))
  return lax.gather(x, idx, dnums, (1,),
                    mode=lax.GatherScatterMode.PROMISE_IN_BOUNDS)


def _merge32(a, b, v):
  vs = _sortd(v)
  m = jnp.maximum(b, lax.rev(vs, (0,)))
  ms = _sortd(m)
  x = jnp.maximum(a, lax.rev(ms, (0,)))
  y = jnp.minimum(a, lax.rev(ms, (0,)))
  return _sortd(x), _sortd(y)


def _maybe_merge(v, carry):
  a, b, t = carry

  def do(_):
    a2, b2 = _merge32(a, b, v)
    t2 = jnp.maximum(t, _splat_lane(b2, LANES - 1))
    return a2, b2, t2

  return lax.cond(_any16(v >= t), do, lambda _: (a, b, t), None)


def _sc_topk(logits):
  mesh = plsc.VectorSubcoreMesh(
      core_axis_name="c", subcore_axis_name="s", num_cores=NC, num_subcores=NS)

  @functools.partial(
      pl.kernel,
      out_type=jax.ShapeDtypeStruct((B, TOPK), jnp.float32),
      mesh=mesh,
      scratch_types=[
          pltpu.VMEM((V,), jnp.float32),
          pltpu.VMEM((NG * LANES,), jnp.float32),
          pltpu.VMEM((NG * LANES + 2 * LANES,), jnp.int32),
          pltpu.VMEM((CAND_CAP + LANES,), jnp.float32),
          pltpu.VMEM((TOPK,), jnp.float32),
          pltpu.SMEM((2,), jnp.int32),
      ],
      compiler_params=pltpu.CompilerParams(needs_layout_passes=False),
  )
  def k(logits_hbm, out_hbm, buf, gmax, wl, cand, obuf, cnt_ref):
    wid = lax.axis_index("s") * NC + lax.axis_index("c")

    def row_body(r, _):
      row = wid * ROWS_PER_W + r
      pltpu.sync_copy(logits_hbm.at[row], buf)

      neg = jnp.full((LANES,), _NEG, jnp.float32)

      @plsc.parallel_loop(0, NG, unroll=2, carry=(neg, neg))
      def seed(g, sc):
        m1, m2 = sc
        vecs = [buf[pl.ds((g * GROUP + j) * LANES, LANES)]
                for j in range(GROUP)]
        while len(vecs) > 1:
          vecs = [jnp.maximum(vecs[i], vecs[i + 1])
                  for i in range(0, len(vecs) - 1, 2)] + (
                      [vecs[-1]] if len(vecs) % 2 else [])
        gmx = vecs[0]
        hi = jnp.maximum(m1, gmx)
        lo = jnp.minimum(m1, gmx)
        gmax[pl.ds(g * LANES, LANES)] = gmx
        return hi, jnp.maximum(m2, lo)

      t0 = jnp.maximum(neg, _splat_lane(_sortd(seed[1]), LANES - 1))

      cnt_ref[0] = 0
      iota = lax.iota(jnp.int32, LANES)

      def compress_one(gv_idx):
        gv = gmax[pl.ds(gv_idx * LANES, LANES)]
        mask = gv >= t0
        ids = gv_idx * LANES + iota
        c = cnt_ref[0]
        plsc.store_compressed(wl.at[pl.ds(c, LANES)], ids, mask=mask)
        cnt_ref[0] = c + plsc.all_reduce_population_count(mask)[0]

      def quad_body(q, _):
        gvi = q * 4
        g0 = gmax[pl.ds(gvi * LANES, LANES)]
        g1 = gmax[pl.ds((gvi + 1) * LANES, LANES)]
        g2 = gmax[pl.ds((gvi + 2) * LANES, LANES)]
        g3 = gmax[pl.ds((gvi + 3) * LANES, LANES)]
        gm = jnp.maximum(jnp.maximum(g0, g1), jnp.maximum(g2, g3))

        def hit(_2):
          for k2 in range(4):
            compress_one(gvi + k2)
          return 0

        return lax.cond(_any16(gm >= t0), hit, lambda _2: 0, 0)

      lax.fori_loop(0, NQUAD, quad_body, 0)
      compress_one(NQUAD * 4)
      n = cnt_ref[0]
      wl[pl.ds(n, LANES)] = jnp.full((LANES,), -1, jnp.int32)

      validc = iota < GROUP
      cnt_ref[1] = 0

      def flush(st):
        c2 = cnt_ref[1]
        cand[pl.ds(c2, LANES)] = jnp.full((LANES,), _NEG, jnp.float32)

        def mb(i, s):
          return _maybe_merge(cand[pl.ds(i * LANES, LANES)], s)

        st = lax.fori_loop(0, (c2 + LANES - 1) // LANES, mb, st)
        cnt_ref[1] = 0
        return st

      def drain_block(i, st):
        st = lax.cond(cnt_ref[1] >= CAND_CAP - LANES * LANES,
                      flush, lambda s: s, st)
        wvec = wl[pl.ds(i * LANES, LANES)]

        for j in range(LANES):
          e = wvec[j]
          g = lax.shift_right_logical(e, 4)
          lane = lax.bitwise_and(e, 15)
          base = g * (GROUP * LANES) + lane
          vmask = jnp.logical_and(validc, e >= 0)
          idx = jnp.where(vmask, base + iota * LANES, 0)
          v = plsc.load_gather(buf, [idx])
          cmask = jnp.logical_and(vmask, v >= t0)
          c2 = cnt_ref[1]
          plsc.store_compressed(cand.at[pl.ds(c2, LANES)], v, mask=cmask)
          cnt_ref[1] = c2 + plsc.all_reduce_population_count(cmask)[0]
        return st

      nblk = (n + LANES - 1) // LANES
      state = lax.fori_loop(0, nblk, drain_block, (neg, neg, t0))
      state = flush(state)

      a, b, _ = state
      obuf[pl.ds(0, LANES)] = a
      obuf[pl.ds(LANES, LANES)] = b
      pltpu.sync_copy(obuf, out_hbm.at[row])
      return 0

    lax.fori_loop(0, ROWS_PER_W, row_body, 0)

  return k(logits)


def _tc_head(x, ln_w, ln_b, w1, b1, w2, b2):
  def body(x_ref, lnw_ref, lnb_ref, w1_ref, b1_ref, w2_ref, b2_ref, o_ref):
    xv = x_ref[...]
    mean = jnp.mean(xv, axis=-1, keepdims=True)
    var = jnp.mean((xv - mean) ** 2, axis=-1, keepdims=True)
    xn = (xv - mean) * lax.rsqrt(var + 1e-5) * lnw_ref[...] + lnb_ref[...]
    h = jnp.dot(xn, w1_ref[...], preferred_element_type=jnp.float32) + b1_ref[...]
    h = 0.5 * h * (1.0 + lax.erf(h * jnp.float32(0.7071067811865476)))
    o_ref[...] = jnp.dot(h, w2_ref[...], preferred_element_type=jnp.float32) + b2_ref[...]

  return pl.pallas_call(
      body,
      out_shape=jax.ShapeDtypeStruct((B, OUT), jnp.float32),
  )(x, ln_w.reshape(1, TOPK), ln_b.reshape(1, TOPK),
    w1, b1.reshape(1, HID), w2, b2.reshape(1, OUT))


def kernel(logits, ln_w, ln_b, W1, b1, W2, b2):
  topk = _sc_topk(logits)
  return _tc_head(topk, ln_w, ln_b, W1, b1, W2, b2)


# speedup vs baseline: 3.0674x; 1.0743x over previous
import functools

import jax
import jax.numpy as jnp
from jax import lax
from jax.experimental import pallas as pl
from jax.experimental.pallas import tpu as pltpu
from jax.experimental.pallas import tpu_sc as plsc

B = 128
V = 100000
TOPK = 32
HID = 128
OUT = 128

NC = 2
NS = 16
NW = NC * NS
ROWS_PER_W = B // NW
LANES = 16

NCHUNK = 0
GROUP = 10
NG = V // (GROUP * LANES)
NQUAD = NG // 4
CAND_CAP = 2048

_NEG = float("-inf")


def _sortd(x):
  return lax.rev(jnp.sort(x), (0,))


def _any16(mask):
  cnt = plsc.all_reduce_population_count(mask)
  return cnt[0] > 0


def _splat_lane(x, lane):
  idx = jnp.full((LANES, 1), lane, jnp.int32)
  dnums = lax.GatherDimensionNumbers(
      offset_dims=(), collapsed_slice_dims=(0,), start_index_map=(0,))
  return lax.gather(x, idx, dnums, (1,),
                    mode=lax.GatherScatterMode.PROMISE_IN_BOUNDS)


def _merge32(a, b, v):
  vs = _sortd(v)
  m = jnp.maximum(b, lax.rev(vs, (0,)))
  ms = _sortd(m)
  x = jnp.maximum(a, lax.rev(ms, (0,)))
  y = jnp.minimum(a, lax.rev(ms, (0,)))
  return _sortd(x), _sortd(y)


def _maybe_merge(v, carry):
  a, b, t = carry

  def do(_):
    a2, b2 = _merge32(a, b, v)
    t2 = jnp.maximum(t, _splat_lane(b2, LANES - 1))
    return a2, b2, t2

  return lax.cond(_any16(v >= t), do, lambda _: (a, b, t), None)


def _sc_topk(logits):
  mesh = plsc.VectorSubcoreMesh(
      core_axis_name="c", subcore_axis_name="s", num_cores=NC, num_subcores=NS)

  @functools.partial(
      pl.kernel,
      out_type=jax.ShapeDtypeStruct((B, TOPK), jnp.float32),
      mesh=mesh,
      scratch_types=[
          pltpu.VMEM((V,), jnp.float32),
          pltpu.VMEM((NG * LANES,), jnp.float32),
          pltpu.VMEM((NG * LANES + 2 * LANES,), jnp.int32),
          pltpu.VMEM((CAND_CAP + LANES,), jnp.float32),
          pltpu.VMEM((TOPK,), jnp.float32),
          pltpu.SMEM((2,), jnp.int32),
      ],
      compiler_params=pltpu.CompilerParams(needs_layout_passes=False),
  )
  def k(logits_hbm, out_hbm, buf, gmax, wl, cand, obuf, cnt_ref):
    wid = lax.axis_index("s") * NC + lax.axis_index("c")

    def row_body(r, _):
      row = wid * ROWS_PER_W + r
      pltpu.sync_copy(logits_hbm.at[row], buf)

      neg = jnp.full((LANES,), _NEG, jnp.float32)

      @plsc.parallel_loop(0, NG, unroll=2, carry=(neg, neg, neg, neg))
      def seed(g, sc):
        m1, m2, m3, m4 = sc
        vecs = [buf[pl.ds((g * GROUP + j) * LANES, LANES)]
                for j in range(GROUP)]
        while len(vecs) > 1:
          vecs = [jnp.maximum(vecs[i], vecs[i + 1])
                  for i in range(0, len(vecs) - 1, 2)] + (
                      [vecs[-1]] if len(vecs) % 2 else [])
        gmx = vecs[0]
        hi1 = jnp.maximum(m1, gmx)
        lo1 = jnp.minimum(m1, gmx)
        hi2 = jnp.maximum(m2, lo1)
        lo2 = jnp.minimum(m2, lo1)
        hi3 = jnp.maximum(m3, lo2)
        lo3 = jnp.minimum(m3, lo2)
        gmax[pl.ds(g * LANES, LANES)] = gmx
        return hi1, hi2, hi3, jnp.maximum(m4, lo3)

      s1, s2, s3, s4 = (_sortd(m) for m in seed)
      a0 = _sortd(jnp.maximum(s1, lax.rev(s2, (0,))))
      b0 = _sortd(jnp.minimum(s1, lax.rev(s2, (0,))))
      for s in (s3, s4):
        ms = _sortd(jnp.maximum(b0, lax.rev(s, (0,))))
        x2 = jnp.maximum(a0, lax.rev(ms, (0,)))
        y2 = jnp.minimum(a0, lax.rev(ms, (0,)))
        a0 = _sortd(x2)
        b0 = _sortd(y2)
      t0 = _splat_lane(b0, LANES - 1)

      cnt_ref[0] = 0
      iota = lax.iota(jnp.int32, LANES)

      def compress_one(gv_idx):
        gv = gmax[pl.ds(gv_idx * LANES, LANES)]
        mask = gv >= t0
        ids = gv_idx * LANES + iota
        c = cnt_ref[0]
        plsc.store_compressed(wl.at[pl.ds(c, LANES)], ids, mask=mask)
        cnt_ref[0] = c + plsc.all_reduce_population_count(mask)[0]

      def quad_body(q, _):
        gvi = q * 4
        g0 = gmax[pl.ds(gvi * LANES, LANES)]
        g1 = gmax[pl.ds((gvi + 1) * LANES, LANES)]
        g2 = gmax[pl.ds((gvi + 2) * LANES, LANES)]
        g3 = gmax[pl.ds((gvi + 3) * LANES, LANES)]
        gm = jnp.maximum(jnp.maximum(g0, g1), jnp.maximum(g2, g3))

        def hit(_2):
          for k2 in range(4):
            compress_one(gvi + k2)
          return 0

        return lax.cond(_any16(gm >= t0), hit, lambda _2: 0, 0)

      lax.fori_loop(0, NQUAD, quad_body, 0)
      compress_one(NQUAD * 4)
      n = cnt_ref[0]
      wl[pl.ds(n, LANES)] = jnp.full((LANES,), -1, jnp.int32)

      validc = iota < GROUP
      cnt_ref[1] = 0

      def flush(st):
        c2 = cnt_ref[1]
        cand[pl.ds(c2, LANES)] = jnp.full((LANES,), _NEG, jnp.float32)

        def mb(i, s):
          return _maybe_merge(cand[pl.ds(i * LANES, LANES)], s)

        st = lax.fori_loop(0, (c2 + LANES - 1) // LANES, mb, st)
        cnt_ref[1] = 0
        return st

      def drain_block(i, st):
        st = lax.cond(cnt_ref[1] >= CAND_CAP - LANES * LANES,
                      flush, lambda s: s, st)
        wvec = wl[pl.ds(i * LANES, LANES)]

        for j in range(LANES):
          e = wvec[j]
          g = lax.shift_right_logical(e, 4)
          lane = lax.bitwise_and(e, 15)
          base = g * (GROUP * LANES) + lane
          vmask = jnp.logical_and(validc, e >= 0)
          idx = jnp.where(vmask, base + iota * LANES, 0)
          v = plsc.load_gather(buf, [idx])
          cmask = jnp.logical_and(vmask, v >= t0)
          c2 = cnt_ref[1]
          plsc.store_compressed(cand.at[pl.ds(c2, LANES)], v, mask=cmask)
          cnt_ref[1] = c2 + plsc.all_reduce_population_count(cmask)[0]
        return st

      nblk = (n + LANES - 1) // LANES
      state = lax.fori_loop(0, nblk, drain_block, (neg, neg, t0))
      state = flush(state)

      a, b, _ = state
      obuf[pl.ds(0, LANES)] = a
      obuf[pl.ds(LANES, LANES)] = b
      pltpu.sync_copy(obuf, out_hbm.at[row])
      return 0

    lax.fori_loop(0, ROWS_PER_W, row_body, 0)

  return k(logits)


def _tc_head(x, ln_w, ln_b, w1, b1, w2, b2):
  def body(x_ref, lnw_ref, lnb_ref, w1_ref, b1_ref, w2_ref, b2_ref, o_ref):
    xv = x_ref[...]
    mean = jnp.mean(xv, axis=-1, keepdims=True)
    var = jnp.mean((xv - mean) ** 2, axis=-1, keepdims=True)
    xn = (xv - mean) * lax.rsqrt(var + 1e-5) * lnw_ref[...] + lnb_ref[...]
    h = jnp.dot(xn, w1_ref[...], preferred_element_type=jnp.float32) + b1_ref[...]
    h = 0.5 * h * (1.0 + lax.erf(h * jnp.float32(0.7071067811865476)))
    o_ref[...] = jnp.dot(h, w2_ref[...], preferred_element_type=jnp.float32) + b2_ref[...]

  return pl.pallas_call(
      body,
      out_shape=jax.ShapeDtypeStruct((B, OUT), jnp.float32),
  )(x, ln_w.reshape(1, TOPK), ln_b.reshape(1, TOPK),
    w1, b1.reshape(1, HID), w2, b2.reshape(1, OUT))


def kernel(logits, ln_w, ln_b, W1, b1, W2, b2):
  topk = _sc_topk(logits)
  return _tc_head(topk, ln_w, ln_b, W1, b1, W2, b2)
